# Initial kernel scaffold; baseline (speedup 1.0000x reference)
import jax
import jax.numpy as jnp
from jax.experimental import pallas as pl


def kernel(src_node_features, neighbor_node_features, W_agg, W_self):
    raise NotImplementedError("write your pallas kernel here")


# speedup vs baseline: 1.3193x; 1.3193x over previous
import jax
import jax.numpy as jnp
from jax.experimental import pallas as pl

_BN = 400


def _body(src_ref, neigh_ref, wa_ref, ws_ref, out_ref):
    mean = jnp.mean(neigh_ref[...], axis=1)
    h = jnp.dot(mean, wa_ref[...], preferred_element_type=jnp.float32)
    h += jnp.dot(src_ref[...], ws_ref[...], preferred_element_type=jnp.float32)
    out_ref[...] = jnp.maximum(h, 0.0)


def kernel(src_node_features, neighbor_node_features, W_agg, W_self):
    n, deg, d_in = neighbor_node_features.shape
    d_hid = W_agg.shape[1]
    grid = (n // _BN,)
    return pl.pallas_call(
        _body,
        grid=grid,
        in_specs=[
            pl.BlockSpec((_BN, d_in), lambda i: (i, 0)),
            pl.BlockSpec((_BN, deg, d_in), lambda i: (i, 0, 0)),
            pl.BlockSpec((d_in, d_hid), lambda i: (0, 0)),
            pl.BlockSpec((d_in, d_hid), lambda i: (0, 0)),
        ],
        out_specs=pl.BlockSpec((_BN, d_hid), lambda i: (i, 0)),
        out_shape=jax.ShapeDtypeStruct((n, d_hid), jnp.float32),
    )(src_node_features, neighbor_node_features, W_agg, W_self)
